# Initial kernel scaffold; baseline (speedup 1.0000x reference)
import jax
import jax.numpy as jnp
from jax.experimental import pallas as pl


def kernel(spec_tensor):
    raise NotImplementedError("write your pallas kernel here")


# speedup vs baseline: 2.7134x; 2.7134x over previous
import functools

import jax
import jax.numpy as jnp
from jax import lax
from jax.experimental import pallas as pl
from jax.experimental.pallas import tpu as pltpu

N_PEAKS = 12000
BLUR_K = 5
BLUR_SIGMA = 1.5


def _dense_body(kb_ref, x_ref, o_ref):
    x = x_ref[0]
    F, T = x.shape
    ninf = jnp.float32(-jnp.inf)
    fidx = lax.broadcasted_iota(jnp.int32, (F, T), 0)
    tidx = lax.broadcasted_iota(jnp.int32, (F, T), 1)

    def peaks(f):
        lt = jnp.where(tidx == T - 1, ninf, jnp.roll(f, -1, axis=1))
        rt = jnp.where(tidx == 0, ninf, jnp.roll(f, 1, axis=1))
        mt = jnp.maximum(f, jnp.maximum(lt, rt))
        lf = jnp.where(fidx == F - 1, ninf, jnp.roll(f, -1, axis=0))
        rf = jnp.where(fidx == 0, ninf, jnp.roll(f, 1, axis=0))
        mf = jnp.maximum(f, jnp.maximum(lf, rf))
        isp = (f == mt) & (f == mf)
        mn = jnp.min(f)
        mx = jnp.max(f)
        fn = (f - mn) / (mx - mn)
        return jnp.where(isp, fn, jnp.float32(0.0))

    def conv_axis(f, axis, n):
        idx = fidx if axis == 0 else tidx
        fb = f.astype(jnp.bfloat16).astype(jnp.float32)
        r1 = jnp.roll(fb, -1, axis)
        rm1 = jnp.roll(fb, 1, axis)
        r2 = jnp.roll(fb, -2, axis)
        rm2 = jnp.roll(fb, 2, axis)
        sm2 = jnp.where(idx == 1, fb, jnp.where(idx == 0, r2, rm2))
        sm1 = jnp.where(idx == 0, r1, rm1)
        sp1 = jnp.where(idx == n - 1, rm1, r1)
        sp2 = jnp.where(idx == n - 2, fb, jnp.where(idx == n - 1, rm2, r2))
        kb = [kb_ref[d].astype(jnp.bfloat16).astype(jnp.float32)
              for d in range(5)]
        acc = kb[0] * sm2
        acc = acc + kb[1] * sm1
        acc = acc + kb[2] * fb
        acc = acc + kb[3] * sp1
        acc = acc + kb[4] * sp2
        return acc

    p1 = peaks(x)
    y = conv_axis(p1, 0, F)
    feat = conv_axis(y, 1, T)
    o_ref[0] = peaks(feat)


def _peak_map(spec_tensor):
    B, F, T = spec_tensor.shape
    half = (BLUR_K - 1) * 0.5
    t = jnp.linspace(-half, half, BLUR_K)
    pdf = jnp.exp(-0.5 * (t / BLUR_SIGMA) ** 2)
    k1 = (pdf / pdf.sum()).astype(jnp.float32)

    return pl.pallas_call(
        _dense_body,
        grid=(B,),
        in_specs=[
            pl.BlockSpec(memory_space=pltpu.SMEM),
            pl.BlockSpec((1, F, T), lambda i: (i, 0, 0)),
        ],
        out_specs=pl.BlockSpec((1, F, T), lambda i: (i, 0, 0)),
        out_shape=jax.ShapeDtypeStruct((B, F, T), jnp.float32),
    )(k1, spec_tensor)


def kernel(spec_tensor):
    B, F, T = spec_tensor.shape
    p = _peak_map(spec_tensor)
    outs = []
    for ix in range(B):
        pi = p[ix]
        fi, ti = jnp.nonzero(pi, size=N_PEAKS, fill_value=0)
        n = jnp.count_nonzero(pi)
        vals = pi[fi, ti]
        pts = jnp.stack([fi.astype(jnp.float32) / F,
                         ti.astype(jnp.float32) / T,
                         vals], axis=1)
        valid = (jnp.arange(N_PEAKS) < n)[:, None]
        pts = jnp.where(valid, pts, jnp.zeros_like(pts))
        outs.append(pts.T)
    return jnp.stack(outs)
